# Initial kernel scaffold; baseline (speedup 1.0000x reference)
import jax
import jax.numpy as jnp
from jax.experimental import pallas as pl


def kernel(x, indices):
    raise NotImplementedError("write your pallas kernel here")


# speedup vs baseline: 1.2911x; 1.2911x over previous
import jax
import jax.numpy as jnp
import numpy as np
from jax import lax
from jax.experimental import pallas as pl
from jax.experimental.pallas import tpu as pltpu
from jax.experimental.pallas import tpu_sc as plsc

OUT_SHAPE = (4, 384, 384, 96)
N = int(np.prod(OUT_SHAPE))
M = int(np.prod((4, 192, 192, 96)))
MB = M // 16
NW = 32
NS = N // NW
W = 8192
WV = W // 16
PV = (W + 16) // 16
ZB = 65536
NZ = NS // ZB
BS_ITERS = int(np.ceil(np.log2(MB + 1)))


def _sc_body(skp_hbm, sv_hbm, out_hbm,
             idx_v, val_v, cidx_v, cval_v, zbuf_v, pr_v, sem):
    wid = lax.axis_index("s") * 2 + lax.axis_index("c")
    lo = wid * NS
    hi = lo + NS

    zero16 = jnp.zeros((16,), jnp.float32)

    def _zb(i, c):
        zbuf_v[pl.ds(i * 16, 16)] = zero16
        return c
    lax.fori_loop(0, ZB // 16, _zb, 0)

    def _zf(c, carry):
        off = pl.multiple_of(lo + c * ZB, 8)
        pltpu.sync_copy(zbuf_v, out_hbm.at[pl.ds(off, ZB)])
        return carry
    lax.fori_loop(0, NZ, _zf, 0)

    def _bsearch(bound):
        def body(_, lbub):
            lb, ub = lbub
            mid = (lb + ub) // 2
            off = pl.multiple_of(mid * 16, 8)
            pltpu.sync_copy(skp_hbm.at[pl.ds(off, 16)], pr_v)
            val = pr_v[...][0]
            pred = val >= bound
            return (jnp.where(pred, lb, mid), jnp.where(pred, mid, ub))
        lb, ub = lax.fori_loop(0, BS_ITERS, body,
                               (jnp.int32(-1), jnp.int32(MB)))
        return ub

    b_lo = _bsearch(lo)
    b_hi = _bsearch(hi)
    start = jnp.maximum(b_lo - 1, 0) * 16
    t_end = b_hi * 16
    nwin = (t_end - start + W - 1) // W

    iota16 = lax.iota(jnp.int32, 16)

    def _window(j, carry):
        base = pl.multiple_of(jnp.minimum(start + j * W, M - W), 8)
        pltpu.sync_copy(skp_hbm.at[pl.ds(base, W + 16)], idx_v)
        pltpu.sync_copy(sv_hbm.at[pl.ds(base, W)], val_v)

        def _vreg(k, cnt):
            v = idx_v[pl.ds(k * 16, 16)]
            vs = idx_v[pl.ds(k * 16 + 1, 16)]
            u = val_v[pl.ds(k * 16, 16)]
            m = (v != vs) & (v >= lo) & (v < hi)
            pc = plsc.cumsum(m.astype(jnp.int32))
            pos = cnt + pc - 1
            plsc.store_scatter(cidx_v, [pos], v, mask=m)
            plsc.store_scatter(cval_v, [pos], u, mask=m)
            return cnt + pc[15]

        cnt = lax.fori_loop(0, WV, _vreg, jnp.int32(0))

        @pl.when(cnt > 0)
        def _fire():
            wi = cidx_v[pl.ds(0, 16)][0]
            wv = cval_v[pl.ds(0, 16)][0]
            wi16 = jnp.full((16,), wi, jnp.int32)
            wv16 = jnp.full((16,), wv, jnp.float32)
            kb = cnt // 16
            pos = kb * 16 + iota16
            mb = pos >= cnt
            vb_i = cidx_v[pl.ds(kb * 16, 16)]
            vb_v = cval_v[pl.ds(kb * 16, 16)]
            cidx_v[pl.ds(kb * 16, 16)] = jnp.where(mb, wi16, vb_i)
            cval_v[pl.ds(kb * 16, 16)] = jnp.where(mb, wv16, vb_v)

            def _pad(k2, c):
                cidx_v[pl.ds(k2 * 16, 16)] = wi16
                cval_v[pl.ds(k2 * 16, 16)] = wv16
                return c
            lax.fori_loop(kb + 1, PV, _pad, 0)
            pltpu.async_copy(cval_v, out_hbm.at[cidx_v], sem).wait()

        return carry

    lax.fori_loop(0, nwin, _window, 0)


@jax.jit
def _scatter(skp, sv):
    mesh = plsc.VectorSubcoreMesh(core_axis_name="c", subcore_axis_name="s",
                                  num_cores=2, num_subcores=16)
    return pl.kernel(
        _sc_body,
        out_type=jax.ShapeDtypeStruct((N,), jnp.float32),
        mesh=mesh,
        compiler_params=pltpu.CompilerParams(needs_layout_passes=False),
        scratch_types=[
            pltpu.VMEM((W + 16,), jnp.int32),
            pltpu.VMEM((W,), jnp.float32),
            pltpu.VMEM((W + 16,), jnp.int32),
            pltpu.VMEM((W + 16,), jnp.float32),
            pltpu.VMEM((ZB,), jnp.float32),
            pltpu.VMEM((16,), jnp.int32),
            pltpu.SemaphoreType.DMA,
        ],
    )(skp, sv)


def kernel(x, indices):
    idx = indices.ravel().astype(jnp.int32)
    keys = jnp.where(idx < 0, idx + N, idx)
    sk, sv = lax.sort((keys, x.ravel()), dimension=0, num_keys=1,
                      is_stable=False)
    skp = jnp.concatenate([sk, jnp.full((16,), -1, jnp.int32)])
    out = _scatter(skp, sv)
    return out.reshape(OUT_SHAPE)


# speedup vs baseline: 3.9834x; 3.0853x over previous
import jax
import jax.numpy as jnp
import numpy as np
from jax import lax
from jax.experimental import pallas as pl
from jax.experimental.pallas import tpu as pltpu
from jax.experimental.pallas import tpu_sc as plsc

OUT_SHAPE = (4, 384, 384, 96)
N = int(np.prod(OUT_SHAPE))
M = int(np.prod((4, 192, 192, 96)))
MB = M // 16
NW = 32
NS = N // NW
W = 8192
WV = W // 16
OB = 65536
NT = NS // OB
BS_ITERS = int(np.ceil(np.log2(MB + 1)))


def _sc_body(skp_hbm, sv_hbm, out_hbm, idx_v, val_v, obuf_v, pr_v):
    wid = lax.axis_index("s") * 2 + lax.axis_index("c")
    lo = wid * NS
    hi = lo + NS

    zero16 = jnp.zeros((16,), jnp.float32)

    def _zero_obuf():
        def _z(i, c):
            obuf_v[pl.ds(i * 16, 16)] = zero16
            return c
        lax.fori_loop(0, OB // 16, _z, 0)

    def _flush(t2):
        off = pl.multiple_of(lo + t2 * OB, 8)
        pltpu.sync_copy(obuf_v, out_hbm.at[pl.ds(off, OB)])
        _zero_obuf()

    def _proc(tlo, tend):
        def _vreg(k, c):
            v = idx_v[pl.ds(k * 16, 16)]
            vs = idx_v[pl.ds(k * 16 + 1, 16)]
            u = val_v[pl.ds(k * 16, 16)]
            m = (v != vs) & (v >= tlo) & (v < tend)
            plsc.store_scatter(obuf_v, [v - tlo], u, mask=m)
            return c
        lax.fori_loop(0, WV, _vreg, 0)

    _zero_obuf()

    def _bsearch(bound):
        def body(_, lbub):
            lb, ub = lbub
            mid = (lb + ub) // 2
            off = pl.multiple_of(mid * 16, 8)
            pltpu.sync_copy(skp_hbm.at[pl.ds(off, 16)], pr_v)
            val = pr_v[...][0]
            pred = val >= bound
            return (jnp.where(pred, lb, mid), jnp.where(pred, mid, ub))
        lb, ub = lax.fori_loop(0, BS_ITERS, body,
                               (jnp.int32(-1), jnp.int32(MB)))
        return ub

    b_lo = _bsearch(lo)
    b_hi = _bsearch(hi)
    start = jnp.maximum(b_lo - 1, 0) * 16
    t_end = b_hi * 16
    nwin = (t_end - start + W - 1) // W

    def _window(j, t):
        base = pl.multiple_of(jnp.minimum(start + j * W, M - W), 8)
        pltpu.sync_copy(skp_hbm.at[pl.ds(base, W + 16)], idx_v)
        pltpu.sync_copy(sv_hbm.at[pl.ds(base, W)], val_v)
        last_v = idx_v[pl.ds(W - 16, 16)][15]

        def _cond(t2):
            return (t2 < NT) & (last_v >= lo + (t2 + 1) * OB)

        def _adv(t2):
            tlo = lo + t2 * OB
            _proc(tlo, tlo + OB)
            _flush(t2)
            return t2 + 1

        t = lax.while_loop(_cond, _adv, t)
        tlo = lo + t * OB
        _proc(tlo, tlo + OB)
        return t

    t = lax.fori_loop(0, nwin, _window, jnp.int32(0))

    def _tail(t2, c):
        _flush(t2)
        return c
    lax.fori_loop(t, NT, _tail, 0)


@jax.jit
def _scatter(skp, sv):
    mesh = plsc.VectorSubcoreMesh(core_axis_name="c", subcore_axis_name="s",
                                  num_cores=2, num_subcores=16)
    return pl.kernel(
        _sc_body,
        out_type=jax.ShapeDtypeStruct((N,), jnp.float32),
        mesh=mesh,
        compiler_params=pltpu.CompilerParams(needs_layout_passes=False),
        scratch_types=[
            pltpu.VMEM((W + 16,), jnp.int32),
            pltpu.VMEM((W,), jnp.float32),
            pltpu.VMEM((OB,), jnp.float32),
            pltpu.VMEM((16,), jnp.int32),
        ],
    )(skp, sv)


def kernel(x, indices):
    idx = indices.ravel().astype(jnp.int32)
    keys = jnp.where(idx < 0, idx + N, idx)
    sk, sv = lax.sort((keys, x.ravel()), dimension=0, num_keys=1,
                      is_stable=False)
    skp = jnp.concatenate([sk, jnp.full((16,), -1, jnp.int32)])
    out = _scatter(skp, sv)
    return out.reshape(OUT_SHAPE)
